# baseline (speedup 1.0000x reference)
import jax
import jax.numpy as jnp
from jax.experimental import pallas as pl
from jax.experimental.pallas import tpu as pltpu

_B, _C, _K, _H, _W = 64, 512, 64, 64, 64
_HW = _H * _W
_R = _HW // _C
_NC = 4
_CC = _C // _NC
_EPS = 1e-12


def _netvlad_kernel(xm_ref, w_ref, b_ref, cent_ref, out_ref,
                    xfu_scr, u_scr, ssq_scr, xc_scr):
    j = pl.program_id(1)

    @pl.when(j == 0)
    def _():
        u_scr[...] = jnp.zeros_like(u_scr)
        ssq_scr[...] = jnp.zeros_like(ssq_scr)

    xc4 = xm_ref[...].reshape(_CC, _H, _W)
    xcb = xc4.astype(jnp.bfloat16)
    xc_scr[...] = xcb.reshape(_CC, _HW)
    xc2 = xc_scr[...]
    xcf = xc2.astype(jnp.float32)
    ssq_scr[...] += jnp.sum((xcf * xcf).reshape(_CC, _R, _C), axis=0)
    u_scr[...] += jax.lax.dot_general(
        w_ref[:, pl.ds(j * _CC, _CC)].astype(jnp.bfloat16), xc2,
        (((1,), (0,)), ((), ())), preferred_element_type=jnp.float32)
    xfu_scr[pl.ds(j * (_CC * _R), _CC * _R), :] = (
        xc2.reshape(_CC, _R, _C).reshape(_CC * _R, _C))

    @pl.when(j == _NC - 1)
    def _():
        rnorm_rc = 1.0 / jnp.maximum(jnp.sqrt(ssq_scr[...]), _EPS)
        rnorm = jnp.concatenate(
            [rnorm_rc[r:r + 1, :] for r in range(_R)], axis=1)
        logits = u_scr[...] * rnorm + b_ref[...]
        m = jnp.max(logits, axis=0, keepdims=True)
        e = jnp.exp(logits - m)
        a = e / jnp.sum(e, axis=0, keepdims=True)

        xfn = (xfu_scr[...].reshape(_C, _R, _C)
               * rnorm_rc[None].astype(jnp.bfloat16)).reshape(_HW, _C)
        vlad = jax.lax.dot_general(
            a.astype(jnp.bfloat16), xfn, (((1,), (0,)), ((), ())),
            preferred_element_type=jnp.float32)
        vlad = vlad - jnp.sum(a, axis=1, keepdims=True) * cent_ref[...]
        n1 = jnp.sqrt(jnp.sum(vlad * vlad, axis=1, keepdims=True))
        vlad = vlad / jnp.maximum(n1, _EPS)
        n2 = jnp.sqrt(jnp.sum(vlad * vlad))
        out_ref[0] = vlad / jnp.maximum(n2, _EPS)


def kernel(x, conv_w, conv_b, centroids):
    xm = x.reshape(_B * _C * _H, _W)
    out = pl.pallas_call(
        _netvlad_kernel,
        grid=(_B, _NC),
        in_specs=[
            pl.BlockSpec((_CC * _H, _W), lambda i, j: (i * _NC + j, 0)),
            pl.BlockSpec((_K, _C), lambda i, j: (0, 0)),
            pl.BlockSpec((_K, 1), lambda i, j: (0, 0)),
            pl.BlockSpec((_K, _C), lambda i, j: (0, 0)),
        ],
        out_specs=pl.BlockSpec((1, _K, _C), lambda i, j: (i, 0, 0)),
        out_shape=jax.ShapeDtypeStruct((_B, _K, _C), jnp.float32),
        scratch_shapes=[
            pltpu.VMEM((_HW, _C), jnp.bfloat16),
            pltpu.VMEM((_K, _HW), jnp.float32),
            pltpu.VMEM((_R, _C), jnp.float32),
            pltpu.VMEM((_CC, _HW), jnp.bfloat16),
        ],
        compiler_params=pltpu.CompilerParams(
            dimension_semantics=("parallel", "arbitrary"),
            vmem_limit_bytes=56 * 1024 * 1024,
        ),
        name="netvlad_fused",
    )(xm, conv_w, conv_b.reshape(_K, 1), centroids)
    return out.reshape(_B, _K * _C)


# speedup vs baseline: 1.2963x; 1.2963x over previous
import jax
import jax.numpy as jnp
from jax.experimental import pallas as pl
from jax.experimental.pallas import tpu as pltpu

_B, _C, _K, _H, _W = 64, 512, 64, 64, 64
_HW = _H * _W
_R = _HW // _C
_EPS = 1e-12


def _netvlad_kernel(x2_ref, w_ref, b_ref, cent_ref, out_ref):
    x2 = x2_ref[0]
    xb = x2.astype(jnp.bfloat16)
    u = jax.lax.dot_general(
        w_ref[...].astype(jnp.bfloat16), xb, (((1,), (0,)), ((), ())),
        preferred_element_type=jnp.float32)
    ssq = jnp.sum(x2 * x2, axis=0, keepdims=True)
    rnorm = 1.0 / jnp.maximum(jnp.sqrt(ssq), _EPS)
    logits = u * rnorm + b_ref[...]
    m = jnp.max(logits, axis=0, keepdims=True)
    e = jnp.exp(logits - m)
    a = e / jnp.sum(e, axis=0, keepdims=True)

    xn_bf = xb * rnorm.astype(jnp.bfloat16)
    xfn = xn_bf.reshape(_C, _R, _C).reshape(_HW, _C)

    vlad = jax.lax.dot_general(
        a.astype(jnp.bfloat16), xfn, (((1,), (0,)), ((), ())),
        preferred_element_type=jnp.float32)
    vlad = vlad - jnp.sum(a, axis=1, keepdims=True) * cent_ref[...]
    n1 = jnp.sqrt(jnp.sum(vlad * vlad, axis=1, keepdims=True))
    vlad = vlad / jnp.maximum(n1, _EPS)
    n2 = jnp.sqrt(jnp.sum(vlad * vlad))
    out_ref[0] = vlad / jnp.maximum(n2, _EPS)


def kernel(x, conv_w, conv_b, centroids):
    x2 = x.reshape(_B, _C, _HW)
    out = pl.pallas_call(
        _netvlad_kernel,
        grid=(_B,),
        in_specs=[
            pl.BlockSpec((1, _C, _HW), lambda i: (i, 0, 0)),
            pl.BlockSpec((_K, _C), lambda i: (0, 0)),
            pl.BlockSpec((_K, 1), lambda i: (0, 0)),
            pl.BlockSpec((_K, _C), lambda i: (0, 0)),
        ],
        out_specs=pl.BlockSpec((1, _K, _C), lambda i: (i, 0, 0)),
        out_shape=jax.ShapeDtypeStruct((_B, _K, _C), jnp.float32),
        compiler_params=pltpu.CompilerParams(
            dimension_semantics=("parallel",),
            vmem_limit_bytes=56 * 1024 * 1024,
        ),
        name="netvlad_fused",
    )(x2, conv_w, conv_b.reshape(_K, 1), centroids)
    return out.reshape(_B, _K * _C)
